# baseline (speedup 1.0000x reference)
import functools

import jax
import jax.numpy as jnp
from jax import lax
from jax.experimental import pallas as pl
from jax.experimental.pallas import tpu as pltpu
from jax.experimental.pallas import tpu_sc as plsc

BATCH = 128
N = 32768
K = 1024
NB = 2048
CAP = 4096
PER_LANE = CAP // 16
NV = N // 16
L = 16
NBLK = 8

_SC_INFO = plsc.get_sparse_core_info()
_NC = _SC_INFO.num_cores
_NS = _SC_INFO.num_subcores
_NW = _NC * _NS
_RPW = BATCH // _NW


def _sort_key(x):
    b = plsc.bitcast(x, jnp.int32)
    return jnp.where(b < 0, b, (~b) & jnp.int32(0x7FFFFFFF))


def _body(scores_hbm, out_hbm, x_v, hist_v, ck_v, ci_v, ck2_v, ci2_v, sem):
    wid = lax.axis_index("s") * _NC + lax.axis_index("c")
    iota = lax.iota(jnp.int32, L)
    occ_cal, _ = plsc.scan_count(jnp.zeros((L,), jnp.int32))
    occ_base = occ_cal - iota

    def do_row(r, _):
        row = wid * _RPW + r
        pltpu.sync_copy(scores_hbm.at[row], x_v)

        @plsc.parallel_loop(0, NB // L, unroll=8)
        def _(i):
            hist_v[pl.ds(i * L, L)] = jnp.zeros((L,), jnp.int32)

        @plsc.parallel_loop(0, CAP // L, unroll=8)
        def _(i):
            ck_v[pl.ds(i * L, L)] = jnp.full((L,), -1, jnp.int32)
            ci_v[pl.ds(i * L, L)] = jnp.full((L,), 32767, jnp.int32)

        @plsc.parallel_loop(0, NV, unroll=8)
        def _(i):
            x = x_v[pl.ds(i * L, L)]
            bkt = lax.shift_right_logical(_sort_key(x), 21)
            occ, lastm = plsc.scan_count(bkt)
            plsc.addupdate_scatter(hist_v, [bkt], occ - occ_base + 1,
                                   mask=lastm)

        @plsc.parallel_loop(
            0, NB // L, unroll=4,
            carry=(jnp.int32(0), jnp.full((L,), 1 << 20, jnp.int32)))
        def bfinal(i, carry):
            tot, acc = carry
            h = hist_v[pl.ds(i * L, L)]
            c = plsc.cumsum(h) + tot
            cand = jnp.where(c >= K, i * L + iota, jnp.int32(1 << 20))
            return jnp.max(c), jnp.minimum(acc, cand)
        bstar = jnp.min(bfinal[1])

        @plsc.parallel_loop(0, NV, unroll=8, carry=jnp.zeros((L,), jnp.int32))
        def off(j, off):
            x = x_v[pl.ds(j * L, L)]
            k = _sort_key(x)
            bkt = lax.shift_right_logical(k, 21)
            m = (bkt <= bstar) & (off < PER_LANE)
            dest = off * L + iota
            plsc.store_scatter(ck_v, [dest], k, mask=m)
            plsc.store_scatter(ci_v, [dest], j * L + iota, mask=m)
            return off + jnp.where(m, 1, 0).astype(jnp.int32)
        nv_sort = jnp.max(off)
        nb = lax.shift_right_logical(nv_sort + (NBLK - 1), 3)

        bufs = ((ck_v, ci_v), (ck2_v, ci2_v))
        for p in range(6):
            src_k, src_i = bufs[p % 2]
            dst_k, dst_i = bufs[(p + 1) % 2]

            def digit_of(k, vi, _p=p):
                if _p == 0:
                    return vi & 0xFF
                if _p == 1:
                    return lax.shift_right_logical(vi, 8) & 0xFF
                return lax.shift_right_logical(k, 8 * (_p - 2)) & 0xFF

            @plsc.parallel_loop(0, NBLK * 256 // L, unroll=8)
            def _(i):
                hist_v[pl.ds(i * L, L)] = jnp.zeros((L,), jnp.int32)

            @plsc.parallel_loop(0, nb, unroll=2)
            def _(i, _src_k=src_k, _src_i=src_i, _dig=digit_of):
                for blk in range(NBLK):
                    v = blk * nb + i
                    vm = jnp.full((L,), v < nv_sort)
                    k = _src_k[pl.ds(v * L, L)]
                    vi = _src_i[pl.ds(v * L, L)]
                    d = _dig(k, vi) + blk * 256
                    occ, lastm = plsc.scan_count(d)
                    plsc.addupdate_scatter(hist_v, [d], occ - occ_base + 1,
                                           mask=lastm & vm)

            def comb(j, tot):
                cs = [hist_v[pl.ds(blk * 256 + j * L, L)]
                      for blk in range(NBLK)]
                partial = [jnp.zeros((L,), jnp.int32)]
                for blk in range(NBLK - 1):
                    partial.append(partial[-1] + cs[blk])
                total = partial[-1] + cs[NBLK - 1]
                ctot = plsc.cumsum(total)
                ex = ctot - total + tot
                for blk in range(NBLK):
                    hist_v[pl.ds(blk * 256 + j * L, L)] = ex + partial[blk]
                return tot + jnp.max(ctot)
            lax.fori_loop(0, 256 // L, comb, jnp.int32(0))

            def rperm(i, _, _src_k=src_k, _src_i=src_i, _dst_k=dst_k,
                      _dst_i=dst_i, _dig=digit_of):
                for blk in range(NBLK):
                    v = blk * nb + i
                    vm = jnp.full((L,), v < nv_sort)
                    k = _src_k[pl.ds(v * L, L)]
                    vi = _src_i[pl.ds(v * L, L)]
                    d = _dig(k, vi) + blk * 256
                    offs = plsc.load_gather(hist_v, [d], mask=vm)
                    occ, lastm = plsc.scan_count(d)
                    occ0 = occ - occ_base
                    dest = offs + occ0
                    plsc.store_scatter(_dst_k, [dest], k, mask=vm)
                    plsc.store_scatter(_dst_i, [dest], vi, mask=vm)
                    plsc.addupdate_scatter(hist_v, [d], occ0 + 1,
                                           mask=lastm & vm)
                return 0
            lax.fori_loop(0, nb, rperm, 0)

        pltpu.sync_copy(ci_v.at[pl.ds(0, K)], out_hbm.at[row])
        return 0

    lax.fori_loop(0, _RPW, do_row, 0)


@functools.partial(
    pl.kernel,
    out_type=jax.ShapeDtypeStruct((BATCH, K), jnp.int32),
    mesh=plsc.VectorSubcoreMesh(core_axis_name="c", subcore_axis_name="s"),
    compiler_params=pltpu.CompilerParams(needs_layout_passes=False),
    scratch_types=[
        pltpu.VMEM((N,), jnp.float32),
        pltpu.VMEM((NBLK * 256,), jnp.int32),
        pltpu.VMEM((CAP,), jnp.int32),
        pltpu.VMEM((CAP,), jnp.int32),
        pltpu.VMEM((CAP,), jnp.int32),
        pltpu.VMEM((CAP,), jnp.int32),
        pltpu.SemaphoreType.DMA,
    ],
)
def _topk_sc(scores_hbm, out_hbm, *rest):
    _body(scores_hbm, out_hbm, *rest)


@jax.jit
def kernel(input):
    scores = jnp.squeeze(input, axis=-1)
    return _topk_sc(scores)


# speedup vs baseline: 1.1013x; 1.1013x over previous
import functools

import jax
import jax.numpy as jnp
from jax import lax
from jax.experimental import pallas as pl
from jax.experimental.pallas import tpu as pltpu
from jax.experimental.pallas import tpu_sc as plsc

BATCH = 128
N = 32768
K = 1024
NB = 2048
CAP = 4096
PER_LANE = CAP // 16
NV = N // 16
L = 16
NBLK = 4

_SC_INFO = plsc.get_sparse_core_info()
_NC = _SC_INFO.num_cores
_NS = _SC_INFO.num_subcores
_NW = _NC * _NS
_RPW = BATCH // _NW


def _sort_key(x):
    b = plsc.bitcast(x, jnp.int32)
    return jnp.where(b < 0, b, (~b) & jnp.int32(0x7FFFFFFF))


def _body(scores_hbm, out_hbm, x_v, hist_v, ck_v, ci_v, ck2_v, ci2_v,
          bh_refs, sem):
    wid = lax.axis_index("s") * _NC + lax.axis_index("c")
    iota = lax.iota(jnp.int32, L)
    occ_cal, _ = plsc.scan_count(jnp.zeros((L,), jnp.int32))
    occ_base = occ_cal - iota

    def do_row(r, _):
        row = wid * _RPW + r
        pltpu.sync_copy(scores_hbm.at[row], x_v)

        @plsc.parallel_loop(0, NB // L, unroll=8)
        def _(i):
            hist_v[pl.ds(i * L, L)] = jnp.zeros((L,), jnp.int32)

        @plsc.parallel_loop(0, CAP // L, unroll=8)
        def _(i):
            ck_v[pl.ds(i * L, L)] = jnp.full((L,), -1, jnp.int32)
            ci_v[pl.ds(i * L, L)] = jnp.full((L,), 32767, jnp.int32)

        @plsc.parallel_loop(0, NV, unroll=8)
        def _(i):
            x = x_v[pl.ds(i * L, L)]
            bkt = lax.shift_right_logical(_sort_key(x), 21)
            occ, lastm = plsc.scan_count(bkt)
            plsc.addupdate_scatter(hist_v, [bkt], occ - occ_base + 1,
                                   mask=lastm)

        @plsc.parallel_loop(
            0, NB // L, unroll=4,
            carry=(jnp.int32(0), jnp.full((L,), 1 << 20, jnp.int32)))
        def bfinal(i, carry):
            tot, acc = carry
            h = hist_v[pl.ds(i * L, L)]
            c = plsc.cumsum(h) + tot
            cand = jnp.where(c >= K, i * L + iota, jnp.int32(1 << 20))
            return jnp.max(c), jnp.minimum(acc, cand)
        bstar = jnp.min(bfinal[1])

        @plsc.parallel_loop(0, NV, unroll=16,
                            carry=jnp.zeros((L,), jnp.int32))
        def off(j, off):
            x = x_v[pl.ds(j * L, L)]
            k = _sort_key(x)
            bkt = lax.shift_right_logical(k, 21)
            m = (bkt <= bstar) & (off < PER_LANE)
            dest = off * L + iota
            plsc.store_scatter(ck_v, [dest], k, mask=m)
            plsc.store_scatter(ci_v, [dest], j * L + iota, mask=m)
            return off + jnp.where(m, 1, 0).astype(jnp.int32)
        nv_sort = jnp.max(off)
        nb = lax.shift_right_logical(nv_sort + (NBLK - 1), 2)

        bufs = ((ck_v, ci_v), (ck2_v, ci2_v))
        for p in range(6):
            src_k, src_i = bufs[p % 2]
            dst_k, dst_i = bufs[(p + 1) % 2]

            def digit_of(k, vi, _p=p):
                if _p == 0:
                    return vi & 0xFF
                if _p == 1:
                    return lax.shift_right_logical(vi, 8) & 0xFF
                return lax.shift_right_logical(k, 8 * (_p - 2)) & 0xFF

            @plsc.parallel_loop(0, 256 // L, unroll=4)
            def _(i):
                for bh in bh_refs:
                    bh[pl.ds(i * L, L)] = jnp.zeros((L,), jnp.int32)

            @plsc.parallel_loop(0, nb, unroll=2)
            def _(i, _src_k=src_k, _src_i=src_i, _dig=digit_of):
                for blk in range(NBLK):
                    v = blk * nb + i
                    vm = jnp.full((L,), v < nv_sort)
                    k = _src_k[pl.ds(v * L, L)]
                    vi = _src_i[pl.ds(v * L, L)]
                    d = _dig(k, vi)
                    occ, lastm = plsc.scan_count(d)
                    plsc.addupdate_scatter(bh_refs[blk], [d],
                                           occ - occ_base + 1,
                                           mask=lastm & vm)

            def comb(j, tot):
                cs = [bh[pl.ds(j * L, L)] for bh in bh_refs]
                partial = [jnp.zeros((L,), jnp.int32)]
                for blk in range(NBLK - 1):
                    partial.append(partial[-1] + cs[blk])
                total = partial[-1] + cs[NBLK - 1]
                ctot = plsc.cumsum(total)
                ex = ctot - total + tot
                for blk in range(NBLK):
                    bh_refs[blk][pl.ds(j * L, L)] = ex + partial[blk]
                return tot + jnp.max(ctot)
            lax.fori_loop(0, 256 // L, comb, jnp.int32(0))

            def rperm(i, _, _src_k=src_k, _src_i=src_i, _dst_k=dst_k,
                      _dst_i=dst_i, _dig=digit_of):
                for blk in range(NBLK):
                    v = blk * nb + i
                    vm = jnp.full((L,), v < nv_sort)
                    k = _src_k[pl.ds(v * L, L)]
                    vi = _src_i[pl.ds(v * L, L)]
                    d = _dig(k, vi)
                    offs = plsc.load_gather(bh_refs[blk], [d], mask=vm)
                    occ, lastm = plsc.scan_count(d)
                    occ0 = occ - occ_base
                    dest = offs + occ0
                    plsc.store_scatter(_dst_k, [dest], k, mask=vm)
                    plsc.store_scatter(_dst_i, [dest], vi, mask=vm)
                    plsc.addupdate_scatter(bh_refs[blk], [d], occ0 + 1,
                                           mask=lastm & vm)
                return 0
            lax.fori_loop(0, nb, rperm, 0)

        pltpu.sync_copy(ci_v.at[pl.ds(0, K)], out_hbm.at[row])
        return 0

    lax.fori_loop(0, _RPW, do_row, 0)


@functools.partial(
    pl.kernel,
    out_type=jax.ShapeDtypeStruct((BATCH, K), jnp.int32),
    mesh=plsc.VectorSubcoreMesh(core_axis_name="c", subcore_axis_name="s"),
    compiler_params=pltpu.CompilerParams(needs_layout_passes=False),
    scratch_types=[
        pltpu.VMEM((N,), jnp.float32),
        pltpu.VMEM((NB,), jnp.int32),
        pltpu.VMEM((CAP,), jnp.int32),
        pltpu.VMEM((CAP,), jnp.int32),
        pltpu.VMEM((CAP,), jnp.int32),
        pltpu.VMEM((CAP,), jnp.int32),
        [pltpu.VMEM((256,), jnp.int32) for _ in range(NBLK)],
        pltpu.SemaphoreType.DMA,
    ],
)
def _topk_sc(scores_hbm, out_hbm, *rest):
    _body(scores_hbm, out_hbm, *rest)


@jax.jit
def kernel(input):
    scores = jnp.squeeze(input, axis=-1)
    return _topk_sc(scores)


# speedup vs baseline: 1.2528x; 1.1376x over previous
import functools

import jax
import jax.numpy as jnp
from jax import lax
from jax.experimental import pallas as pl
from jax.experimental.pallas import tpu as pltpu
from jax.experimental.pallas import tpu_sc as plsc

BATCH = 128
N = 32768
K = 1024
NB = 2048
CAP = 4096
PER_LANE = CAP // 16
NV = N // 16
L = 16

_SC_INFO = plsc.get_sparse_core_info()
_NC = _SC_INFO.num_cores
_NS = _SC_INFO.num_subcores
_NW = _NC * _NS
_RPW = BATCH // _NW


def _sort_key(x):
    b = plsc.bitcast(x, jnp.int32)
    return jnp.where(b < 0, b, (~b) & jnp.int32(0x7FFFFFFF))


def _body(scores_hbm, out_hbm, x_v, hist_v, ck_v, ci_v, ck2_v, ci2_v,
          rhist_v, sem):
    wid = lax.axis_index("s") * _NC + lax.axis_index("c")
    iota = lax.iota(jnp.int32, L)
    occ_cal, _ = plsc.scan_count(jnp.zeros((L,), jnp.int32))
    occ_base = occ_cal - iota

    def do_row(r, _):
        row = wid * _RPW + r
        pltpu.sync_copy(scores_hbm.at[row], x_v)

        @plsc.parallel_loop(0, NB // L, unroll=8)
        def _(i):
            hist_v[pl.ds(i * L, L)] = jnp.zeros((L,), jnp.int32)

        @plsc.parallel_loop(0, CAP // L, unroll=8)
        def _(i):
            ck_v[pl.ds(i * L, L)] = jnp.full((L,), -1, jnp.int32)
            ci_v[pl.ds(i * L, L)] = jnp.full((L,), 32767, jnp.int32)

        @plsc.parallel_loop(0, NV, unroll=8)
        def _(i):
            x = x_v[pl.ds(i * L, L)]
            bkt = lax.shift_right_logical(_sort_key(x), 21)
            occ, lastm = plsc.scan_count(bkt)
            plsc.addupdate_scatter(hist_v, [bkt], occ - occ_base + 1,
                                   mask=lastm)

        @plsc.parallel_loop(
            0, NB // L, unroll=4,
            carry=(jnp.int32(0), jnp.full((L,), 1 << 20, jnp.int32)))
        def bfinal(i, carry):
            tot, acc = carry
            h = hist_v[pl.ds(i * L, L)]
            c = plsc.cumsum(h) + tot
            cand = jnp.where(c >= K, i * L + iota, jnp.int32(1 << 20))
            return jnp.max(c), jnp.minimum(acc, cand)
        bstar = jnp.min(bfinal[1])

        @plsc.parallel_loop(0, NV, unroll=16,
                            carry=jnp.zeros((L,), jnp.int32))
        def off(j, off):
            x = x_v[pl.ds(j * L, L)]
            k = _sort_key(x)
            bkt = lax.shift_right_logical(k, 21)
            m = (bkt <= bstar) & (off < PER_LANE)
            dest = off * L + iota
            plsc.store_scatter(ck_v, [dest], k, mask=m)
            plsc.store_scatter(ci_v, [dest], j * L + iota, mask=m)
            return off + jnp.where(m, 1, 0).astype(jnp.int32)
        nv_sort = jnp.max(off)

        bufs = ((ck_v, ci_v), (ck2_v, ci2_v))
        for p in range(6):
            src_k, src_i = bufs[p % 2]
            dst_k, dst_i = bufs[(p + 1) % 2]

            def digit_of(k, vi, _p=p):
                if _p == 0:
                    return vi & 0xFF
                if _p == 1:
                    return lax.shift_right_logical(vi, 8) & 0xFF
                return lax.shift_right_logical(k, 8 * (_p - 2)) & 0xFF

            @plsc.parallel_loop(0, 256 // L, unroll=8)
            def _(i):
                rhist_v[pl.ds(i * L, L)] = jnp.zeros((L,), jnp.int32)

            @plsc.parallel_loop(0, nv_sort, unroll=4)
            def _(i, _src_k=src_k, _src_i=src_i, _dig=digit_of):
                k = _src_k[pl.ds(i * L, L)]
                vi = _src_i[pl.ds(i * L, L)]
                d = _dig(k, vi)
                occ, lastm = plsc.scan_count(d)
                plsc.addupdate_scatter(rhist_v, [d], occ - occ_base + 1,
                                       mask=lastm)

            @plsc.parallel_loop(0, 256 // L, unroll=4, carry=jnp.int32(0))
            def _(i, tot):
                h = rhist_v[pl.ds(i * L, L)]
                c = plsc.cumsum(h)
                rhist_v[pl.ds(i * L, L)] = c - h + tot
                return tot + jnp.max(c)

            def rperm(i, _, _src_k=src_k, _src_i=src_i, _dst_k=dst_k,
                      _dst_i=dst_i, _dig=digit_of):
                k = _src_k[pl.ds(i * L, L)]
                vi = _src_i[pl.ds(i * L, L)]
                d = _dig(k, vi)
                offs = plsc.load_gather(rhist_v, [d])
                occ, lastm = plsc.scan_count(d)
                occ0 = occ - occ_base
                dest = offs + occ0
                plsc.store_scatter(_dst_k, [dest], k)
                plsc.store_scatter(_dst_i, [dest], vi)
                plsc.addupdate_scatter(rhist_v, [d], occ0 + 1, mask=lastm)
                return 0
            lax.fori_loop(0, nv_sort, rperm, 0)

        pltpu.sync_copy(ci_v.at[pl.ds(0, K)], out_hbm.at[row])
        return 0

    lax.fori_loop(0, _RPW, do_row, 0)


@functools.partial(
    pl.kernel,
    out_type=jax.ShapeDtypeStruct((BATCH, K), jnp.int32),
    mesh=plsc.VectorSubcoreMesh(core_axis_name="c", subcore_axis_name="s"),
    compiler_params=pltpu.CompilerParams(needs_layout_passes=False),
    scratch_types=[
        pltpu.VMEM((N,), jnp.float32),
        pltpu.VMEM((NB,), jnp.int32),
        pltpu.VMEM((CAP,), jnp.int32),
        pltpu.VMEM((CAP,), jnp.int32),
        pltpu.VMEM((CAP,), jnp.int32),
        pltpu.VMEM((CAP,), jnp.int32),
        pltpu.VMEM((256,), jnp.int32),
        pltpu.SemaphoreType.DMA,
    ],
)
def _topk_sc(scores_hbm, out_hbm, *rest):
    _body(scores_hbm, out_hbm, *rest)


@jax.jit
def kernel(input):
    scores = jnp.squeeze(input, axis=-1)
    return _topk_sc(scores)


# speedup vs baseline: 1.3085x; 1.0444x over previous
import functools

import jax
import jax.numpy as jnp
from jax import lax
from jax.experimental import pallas as pl
from jax.experimental.pallas import tpu as pltpu
from jax.experimental.pallas import tpu_sc as plsc

BATCH = 128
N = 32768
K = 1024
NB = 2048
CAP = 4096
PER_LANE = CAP // 16
NV = N // 16
L = 16

_SC_INFO = plsc.get_sparse_core_info()
_NC = _SC_INFO.num_cores
_NS = _SC_INFO.num_subcores
_NW = _NC * _NS
_RPW = BATCH // _NW


def _sort_key(x):
    b = plsc.bitcast(x, jnp.int32)
    return jnp.where(b < 0, b, (~b) & jnp.int32(0x7FFFFFFF))


def _body(scores_hbm, out_hbm, x_v, hist_v, ck_v, ci_v, ck2_v, ci2_v,
          rhist_v, sem):
    wid = lax.axis_index("s") * _NC + lax.axis_index("c")
    iota = lax.iota(jnp.int32, L)
    occ_cal, _ = plsc.scan_count(jnp.zeros((L,), jnp.int32))
    occ_base = occ_cal - iota

    row0 = wid * _RPW
    pltpu.async_copy(scores_hbm.at[row0], x_v, sem)

    def do_row(r, _):
        row = row0 + r
        pltpu.make_async_copy(scores_hbm.at[row], x_v, sem).wait()

        @plsc.parallel_loop(0, NB // L, unroll=8)
        def _(i):
            hist_v[pl.ds(i * L, L)] = jnp.zeros((L,), jnp.int32)

        @plsc.parallel_loop(0, CAP // L, unroll=8)
        def _(i):
            ck_v[pl.ds(i * L, L)] = jnp.full((L,), -1, jnp.int32)
            ci_v[pl.ds(i * L, L)] = jnp.full((L,), 32767, jnp.int32)

        @plsc.parallel_loop(0, NV, unroll=8)
        def _(i):
            x = x_v[pl.ds(i * L, L)]
            bkt = lax.shift_right_logical(_sort_key(x), 21)
            occ, lastm = plsc.scan_count(bkt)
            plsc.addupdate_scatter(hist_v, [bkt], occ - occ_base + 1,
                                   mask=lastm)

        @plsc.parallel_loop(
            0, NB // L, unroll=4,
            carry=(jnp.int32(0), jnp.full((L,), 1 << 20, jnp.int32)))
        def bfinal(i, carry):
            tot, acc = carry
            h = hist_v[pl.ds(i * L, L)]
            c = plsc.cumsum(h) + tot
            cand = jnp.where(c >= K, i * L + iota, jnp.int32(1 << 20))
            return jnp.max(c), jnp.minimum(acc, cand)
        bstar = jnp.min(bfinal[1])

        @plsc.parallel_loop(0, NV, unroll=16,
                            carry=jnp.zeros((L,), jnp.int32))
        def off(j, off):
            x = x_v[pl.ds(j * L, L)]
            k = _sort_key(x)
            bkt = lax.shift_right_logical(k, 21)
            m = (bkt <= bstar) & (off < PER_LANE)
            dest = off * L + iota
            plsc.store_scatter(ck_v, [dest], k, mask=m)
            plsc.store_scatter(ci_v, [dest], j * L + iota, mask=m)
            return off + jnp.where(m, 1, 0).astype(jnp.int32)
        nv_sort = jnp.max(off)

        nxt = jnp.minimum(r + 1, _RPW - 1)
        pltpu.async_copy(scores_hbm.at[row0 + nxt], x_v, sem)

        bufs = ((ck_v, ci_v), (ck2_v, ci2_v))
        for p in range(6):
            src_k, src_i = bufs[p % 2]
            dst_k, dst_i = bufs[(p + 1) % 2]

            def digit_of(k, vi, _p=p):
                if _p == 0:
                    return vi & 0xFF
                if _p == 1:
                    return lax.shift_right_logical(vi, 8) & 0xFF
                return lax.shift_right_logical(k, 8 * (_p - 2)) & 0xFF

            @plsc.parallel_loop(0, 256 // L, unroll=8)
            def _(i):
                rhist_v[pl.ds(i * L, L)] = jnp.zeros((L,), jnp.int32)

            @plsc.parallel_loop(0, nv_sort, unroll=4)
            def _(i, _src_k=src_k, _src_i=src_i, _dig=digit_of):
                k = _src_k[pl.ds(i * L, L)]
                vi = _src_i[pl.ds(i * L, L)]
                d = _dig(k, vi)
                occ, lastm = plsc.scan_count(d)
                plsc.addupdate_scatter(rhist_v, [d], occ - occ_base + 1,
                                       mask=lastm)

            @plsc.parallel_loop(0, 256 // L, unroll=4, carry=jnp.int32(0))
            def _(i, tot):
                h = rhist_v[pl.ds(i * L, L)]
                c = plsc.cumsum(h)
                rhist_v[pl.ds(i * L, L)] = c - h + tot
                return tot + jnp.max(c)

            def rperm(i, _, _src_k=src_k, _src_i=src_i, _dst_k=dst_k,
                      _dst_i=dst_i, _dig=digit_of):
                k = _src_k[pl.ds(i * L, L)]
                vi = _src_i[pl.ds(i * L, L)]
                d = _dig(k, vi)
                offs = plsc.load_gather(rhist_v, [d])
                occ, lastm = plsc.scan_count(d)
                occ0 = occ - occ_base
                dest = offs + occ0
                plsc.store_scatter(_dst_k, [dest], k)
                plsc.store_scatter(_dst_i, [dest], vi)
                plsc.addupdate_scatter(rhist_v, [d], occ0 + 1, mask=lastm)
                return 0
            lax.fori_loop(0, nv_sort, rperm, 0)

        pltpu.sync_copy(ci_v.at[pl.ds(0, K)], out_hbm.at[row])
        return 0

    lax.fori_loop(0, _RPW, do_row, 0)
    pltpu.make_async_copy(scores_hbm.at[row0], x_v, sem).wait()


@functools.partial(
    pl.kernel,
    out_type=jax.ShapeDtypeStruct((BATCH, K), jnp.int32),
    mesh=plsc.VectorSubcoreMesh(core_axis_name="c", subcore_axis_name="s"),
    compiler_params=pltpu.CompilerParams(needs_layout_passes=False),
    scratch_types=[
        pltpu.VMEM((N,), jnp.float32),
        pltpu.VMEM((NB,), jnp.int32),
        pltpu.VMEM((CAP,), jnp.int32),
        pltpu.VMEM((CAP,), jnp.int32),
        pltpu.VMEM((CAP,), jnp.int32),
        pltpu.VMEM((CAP,), jnp.int32),
        pltpu.VMEM((256,), jnp.int32),
        pltpu.SemaphoreType.DMA,
    ],
)
def _topk_sc(scores_hbm, out_hbm, *rest):
    _body(scores_hbm, out_hbm, *rest)


@jax.jit
def kernel(input):
    scores = jnp.squeeze(input, axis=-1)
    return _topk_sc(scores)
